# baseline (speedup 1.0000x reference)
import functools

import jax
import jax.numpy as jnp
from jax import lax
from jax.experimental import pallas as pl
from jax.experimental.pallas import tpu as pltpu
from jax.experimental.pallas import tpu_sc as plsc

NUM_POSITIONS = 8192
DIM = 1024
BATCH = 4
SEQ_LEN = 8192

ROWS = BATCH * SEQ_LEN
NC, NS, L = 2, 16, 16
NW = NC * NS
ROWS_PER_W = ROWS // NW
CHUNK = 32
N_CHUNKS = ROWS_PER_W // CHUNK


def _make_sc_kernel():
    mesh = plsc.VectorSubcoreMesh(core_axis_name="c", subcore_axis_name="s")

    @functools.partial(
        pl.kernel,
        mesh=mesh,
        out_type=jax.ShapeDtypeStruct((ROWS, DIM), jnp.float32),
        scratch_types=[
            pltpu.VMEM((CHUNK,), jnp.int32),
            pltpu.VMEM((CHUNK, DIM), jnp.float32),
            pltpu.VMEM((CHUNK, DIM), jnp.float32),
            pltpu.SemaphoreType.DMA,
        ],
    )
    def k(x_hbm, pos_hbm, table_hbm, out_hbm, idx_v, xb, rows_v, sem):
        wid = lax.axis_index("s") * NC + lax.axis_index("c")
        base = wid * ROWS_PER_W

        def chunk_body(i, carry):
            r0 = base + i * CHUNK
            pltpu.sync_copy(pos_hbm.at[pl.ds(r0, CHUNK)], idx_v)
            pltpu.sync_copy(x_hbm.at[pl.ds(r0, CHUNK)], xb)
            pltpu.async_copy(table_hbm.at[idx_v], rows_v, sem).wait()

            def add_row(r, c2):
                for c in range(DIM // L):
                    sl = pl.ds(c * L, L)
                    xb[r, sl] = xb[r, sl] + rows_v[r, sl]
                return c2

            lax.fori_loop(0, CHUNK, add_row, 0)
            pltpu.sync_copy(xb, out_hbm.at[pl.ds(r0, CHUNK)])
            return carry

        lax.fori_loop(0, N_CHUNKS, chunk_body, 0)

    return k


_sc_kernel = _make_sc_kernel()


@jax.jit
def kernel(x, positions, table):
    B, S, D = x.shape
    xf = x.reshape(B * S, D)
    pf = positions.reshape(B * S).astype(jnp.int32)
    out = _sc_kernel(xf, pf, table)
    return out.reshape(B, S, D)


# speedup vs baseline: 1.9164x; 1.9164x over previous
import functools

import jax
import jax.numpy as jnp
from jax import lax
from jax.experimental import pallas as pl
from jax.experimental.pallas import tpu as pltpu
from jax.experimental.pallas import tpu_sc as plsc

NUM_POSITIONS = 8192
DIM = 1024
BATCH = 4
SEQ_LEN = 8192

ROWS = BATCH * SEQ_LEN
NC, NS, L = 2, 16, 16
NW = NC * NS
ROWS_PER_W = ROWS // NW
CHUNK = 16
N_CHUNKS = ROWS_PER_W // CHUNK
NBUF = 2


def _make_sc_kernel():
    mesh = plsc.VectorSubcoreMesh(core_axis_name="c", subcore_axis_name="s")

    row_buf = pltpu.VMEM((CHUNK, DIM), jnp.float32)
    @functools.partial(
        pl.kernel,
        mesh=mesh,
        out_type=jax.ShapeDtypeStruct((ROWS, DIM), jnp.float32),
        scratch_types=(
            [pltpu.VMEM((ROWS_PER_W,), jnp.int32)]
            + [row_buf] * (3 * NBUF)
            + [pltpu.SemaphoreType.DMA] * (3 * NBUF)
        ),
    )
    def k(x_hbm, pos_hbm, table_hbm, out_hbm, idx_all,
          xb0, xb1, gb0, gb1, ob0, ob1,
          sx0, sx1, sg0, sg1, so0, so1):
        xb = [xb0, xb1]
        gb = [gb0, gb1]
        ob = [ob0, ob1]
        sx = [sx0, sx1]
        sg = [sg0, sg1]
        so = [so0, so1]

        wid = lax.axis_index("s") * NC + lax.axis_index("c")
        base = wid * ROWS_PER_W
        pltpu.sync_copy(pos_hbm.at[pl.ds(base, ROWS_PER_W)], idx_all)

        def x_copy(i, b):
            return pltpu.make_async_copy(
                x_hbm.at[pl.ds(base + i * CHUNK, CHUNK)], xb[b], sx[b])

        def g_copy(i, b):
            return pltpu.make_async_copy(
                table_hbm.at[idx_all.at[pl.ds(i * CHUNK, CHUNK)]], gb[b], sg[b])

        def o_copy(i, b):
            return pltpu.make_async_copy(
                ob[b], out_hbm.at[pl.ds(base + i * CHUNK, CHUNK)], so[b])

        for b in range(NBUF):
            x_copy(b, b).start()
            g_copy(b, b).start()

        def group(g_idx, carry):
            for b in range(NBUF):
                i = g_idx * NBUF + b
                x_copy(i, b).wait()
                g_copy(i, b).wait()

                @pl.when(i >= NBUF)
                def _():
                    o_copy(i - NBUF, b).wait()

                def add_row(r, c2):
                    for c in range(DIM // L):
                        sl = pl.ds(c * L, L)
                        ob[b][r, sl] = xb[b][r, sl] + gb[b][r, sl]
                    return c2

                lax.fori_loop(0, CHUNK, add_row, 0)
                o_copy(i, b).start()

                @pl.when(i + NBUF < N_CHUNKS)
                def _():
                    x_copy(i + NBUF, b).start()
                    g_copy(i + NBUF, b).start()
            return carry

        lax.fori_loop(0, N_CHUNKS // NBUF, group, 0)

        for b in range(NBUF):
            o_copy(N_CHUNKS - NBUF + b, b).wait()

    return k


_sc_kernel = _make_sc_kernel()


@jax.jit
def kernel(x, positions, table):
    B, S, D = x.shape
    xf = x.reshape(B * S, D)
    pf = positions.reshape(B * S).astype(jnp.int32)
    out = _sc_kernel(xf, pf, table)
    return out.reshape(B, S, D)


# speedup vs baseline: 1.9857x; 1.0362x over previous
import functools

import jax
import jax.numpy as jnp
from jax import lax
from jax.experimental import pallas as pl
from jax.experimental.pallas import tpu as pltpu
from jax.experimental.pallas import tpu_sc as plsc

NUM_POSITIONS = 8192
DIM = 1024
BATCH = 4
SEQ_LEN = 8192

ROWS = BATCH * SEQ_LEN
NC, NS, L = 2, 16, 16
NW = NC * NS
ROWS_PER_W = ROWS // NW
CHUNK = 8
N_CHUNKS = ROWS_PER_W // CHUNK
NBUF = 3


def _make_sc_kernel():
    mesh = plsc.VectorSubcoreMesh(core_axis_name="c", subcore_axis_name="s")

    row_buf = pltpu.VMEM((CHUNK, DIM), jnp.float32)
    @functools.partial(
        pl.kernel,
        mesh=mesh,
        out_type=jax.ShapeDtypeStruct((ROWS, DIM), jnp.float32),
        scratch_types=(
            [pltpu.VMEM((ROWS_PER_W,), jnp.int32)]
            + [row_buf] * (3 * NBUF)
            + [pltpu.SemaphoreType.DMA] * (3 * NBUF)
        ),
    )
    def k(x_hbm, pos_hbm, table_hbm, out_hbm, idx_all, *scratch):
        bufs, sems = scratch[:3 * NBUF], scratch[3 * NBUF:]
        xb = list(bufs[0:NBUF])
        gb = list(bufs[NBUF:2 * NBUF])
        ob = list(bufs[2 * NBUF:3 * NBUF])
        sx = list(sems[0:NBUF])
        sg = list(sems[NBUF:2 * NBUF])
        so = list(sems[2 * NBUF:3 * NBUF])

        wid = lax.axis_index("s") * NC + lax.axis_index("c")
        base = wid * ROWS_PER_W
        pltpu.sync_copy(pos_hbm.at[pl.ds(base, ROWS_PER_W)], idx_all)

        def x_copy(i, b):
            return pltpu.make_async_copy(
                x_hbm.at[pl.ds(base + i * CHUNK, CHUNK)], xb[b], sx[b])

        def g_copy(i, b):
            return pltpu.make_async_copy(
                table_hbm.at[idx_all.at[pl.ds(i * CHUNK, CHUNK)]], gb[b], sg[b])

        def o_copy(i, b):
            return pltpu.make_async_copy(
                ob[b], out_hbm.at[pl.ds(base + i * CHUNK, CHUNK)], so[b])

        for b in range(NBUF):
            x_copy(b, b).start()
            g_copy(b, b).start()

        def group(g_idx, carry):
            for b in range(NBUF):
                i = g_idx * NBUF + b
                x_copy(i, b).wait()
                g_copy(i, b).wait()

                @pl.when(i >= NBUF)
                def _():
                    o_copy(i - NBUF, b).wait()

                def add_row(r, c2):
                    for c in range(DIM // L):
                        sl = pl.ds(c * L, L)
                        ob[b][r, sl] = xb[b][r, sl] + gb[b][r, sl]
                    return c2

                lax.fori_loop(0, CHUNK, add_row, 0)
                o_copy(i, b).start()

                @pl.when(i + NBUF < N_CHUNKS)
                def _():
                    x_copy(i + NBUF, b).start()
                    g_copy(i + NBUF, b).start()
            return carry

        lax.fori_loop(0, N_CHUNKS // NBUF, group, 0)

        for b in range(NBUF):
            o_copy(N_CHUNKS - NBUF + b, b).wait()

    return k


_sc_kernel = _make_sc_kernel()


@jax.jit
def kernel(x, positions, table):
    B, S, D = x.shape
    xf = x.reshape(B * S, D)
    pf = positions.reshape(B * S).astype(jnp.int32)
    out = _sc_kernel(xf, pf, table)
    return out.reshape(B, S, D)
